# baseline (speedup 1.0000x reference)
import functools

import jax
import jax.numpy as jnp
from jax import lax
from jax.experimental import pallas as pl
from jax.experimental.pallas import tpu as pltpu, tpu_sc as plsc

N_EDGES = 160000
DIM = 256
NSEG = 10000
HALF = DIM // 2

BLK = 1280
CH = 80
N_TILES = 16
EDGES_PER_TILE = N_EDGES // N_TILES
N_CHUNKS = EDGES_PER_TILE // CH

ZROWS_BIG = 640
ZROWS_LAST = NSEG - 15 * ZROWS_BIG


def _tc_body(x_ref, a_ref, wvt_ref, c_ref, bv_ref, d_ref, o_ref):
    x = x_ref[...]
    xb = x.astype(jnp.bfloat16)
    t = jnp.dot(xb, a_ref[...], preferred_element_type=jnp.float32)
    v = lax.dot_general(xb, wvt_ref[...], (((1,), (1,)), ((), ())),
                        preferred_element_type=jnp.float32)
    s = jnp.sum(x * (t + c_ref[...]), axis=1, keepdims=True) + d_ref[...]
    w = 1.0 / (1.0 + jnp.exp(-s))
    msg = (v + bv_ref[...]) * w
    o_ref[0] = msg[:, :HALF]
    o_ref[1] = msg[:, HALF:]


def _edge_messages(x, A, WvT, c, bv, d0):
    return pl.pallas_call(
        _tc_body,
        grid=(N_EDGES // BLK,),
        in_specs=[
            pl.BlockSpec((BLK, DIM), lambda i: (i, 0)),
            pl.BlockSpec((DIM, DIM), lambda i: (0, 0)),
            pl.BlockSpec((DIM, DIM), lambda i: (0, 0)),
            pl.BlockSpec((1, DIM), lambda i: (0, 0)),
            pl.BlockSpec((1, DIM), lambda i: (0, 0)),
            pl.BlockSpec((1, 1), lambda i: (0, 0)),
        ],
        out_specs=pl.BlockSpec((2, BLK, HALF), lambda i: (0, i, 0)),
        out_shape=jax.ShapeDtypeStruct((2, N_EDGES, HALF), jnp.float32),
    )(x, A, WvT, c, bv, d0)


def _sc_body(msg_hbm, idx_hbm, out_hbm, idx_all, msg_v, acc, sem0, sem1):
    cid = lax.axis_index("c")
    sid = lax.axis_index("s")
    col0 = cid * HALF
    base_row = sid * ZROWS_BIG
    base_e = cid * N_EDGES + sid * EDGES_PER_TILE

    def zero_vmem(i, _):
        r = i // (HALF // 16)
        c0 = (i % (HALF // 16)) * 16
        msg_v[0, r, pl.ds(c0, 16)] = jnp.zeros((16,), jnp.float32)
        return 0
    lax.fori_loop(0, CH * (HALF // 16), zero_vmem, 0)

    my_rows = jnp.where(sid < 15, ZROWS_BIG, ZROWS_LAST)

    def zero_acc(i, _):
        pltpu.sync_copy(msg_v.at[0], acc.at[pl.ds(base_row + i * CH, CH), :])
        return 0
    lax.fori_loop(0, my_rows // CH, zero_acc, 0)

    pltpu.sync_copy(idx_hbm.at[sid], idx_all)

    def clamp(i, _):
        r = i // (CH // 16)
        c0 = (i % (CH // 16)) * 16
        idx_all[r, pl.ds(c0, 16)] = jnp.minimum(
            idx_all[r, pl.ds(c0, 16)], NSEG - 1)
        return 0
    lax.fori_loop(0, N_CHUNKS * (CH // 16), clamp, 0)

    plsc.subcore_barrier()

    def start(j, buf, sem):
        return pltpu.async_copy(
            msg_hbm.at[pl.ds(base_e + j * CH, CH), :], msg_v.at[buf], sem)

    def wait(j, buf, sem):
        pltpu.make_async_copy(
            msg_hbm.at[pl.ds(base_e + j * CH, CH), :], msg_v.at[buf], sem
        ).wait()

    def scatter(j, buf):
        pltpu.sync_copy(msg_v.at[buf], acc.at[idx_all.at[j]], add=True)

    start(0, 0, sem0)

    def pair(p, _):
        j0 = 2 * p
        start(j0 + 1, 1, sem1)
        wait(j0, 0, sem0)
        scatter(j0, 0)
        start(j0 + 2, 0, sem0)
        wait(j0 + 1, 1, sem1)
        scatter(j0 + 1, 1)
        return 0
    lax.fori_loop(0, (N_CHUNKS - 1) // 2, pair, 0)

    wait(N_CHUNKS - 1, 0, sem0)
    scatter(N_CHUNKS - 1, 0)

    plsc.subcore_barrier()

    @pl.when(sid < 15)
    def _():
        pltpu.sync_copy(
            acc.at[pl.ds(base_row, ZROWS_BIG), :],
            out_hbm.at[pl.ds(base_row, ZROWS_BIG), pl.ds(col0, HALF)],
        )

    @pl.when(sid == 15)
    def _():
        pltpu.sync_copy(
            acc.at[pl.ds(base_row, ZROWS_LAST), :],
            out_hbm.at[pl.ds(base_row, ZROWS_LAST), pl.ds(col0, HALF)],
        )


def _segment_sum(msg2, idx):
    mesh = plsc.VectorSubcoreMesh(core_axis_name="c", subcore_axis_name="s")

    @functools.partial(
        pl.kernel,
        mesh=mesh,
        out_type=jax.ShapeDtypeStruct((NSEG, DIM), jnp.float32),
        scratch_types=[
            pltpu.VMEM((N_CHUNKS, CH), jnp.int32),
            pltpu.VMEM((2, CH, HALF), jnp.float32),
            pltpu.VMEM_SHARED((NSEG, HALF), jnp.float32),
            pltpu.SemaphoreType.DMA,
            pltpu.SemaphoreType.DMA,
        ],
    )
    def run(msg_hbm, idx_hbm, out_hbm, idx_all, msg_v, acc, sem0, sem1):
        _sc_body(msg_hbm, idx_hbm, out_hbm, idx_all, msg_v, acc, sem0, sem1)

    return run(msg2, idx)


def kernel(x, index, dim_size, Wq, bq, Wk, bk, Wv, bv):
    A = lax.dot_general(Wq, Wk, (((0,), (0,)), ((), ())))
    c = (Wq.T @ bk + Wk.T @ bq).reshape(1, DIM)
    d0 = jnp.dot(bq, bk).reshape(1, 1)
    bv2 = bv.reshape(1, DIM)

    msg = _edge_messages(x, A.astype(jnp.bfloat16), Wv.astype(jnp.bfloat16),
                         c, bv2, d0)
    msg2 = msg.reshape(2 * N_EDGES, HALF)

    idx3d = index.astype(jnp.int32).reshape(N_TILES, N_CHUNKS, CH)
    return _segment_sum(msg2, idx3d)


# speedup vs baseline: 1.1190x; 1.1190x over previous
import functools

import jax
import jax.numpy as jnp
from jax import lax
from jax.experimental import pallas as pl
from jax.experimental.pallas import tpu as pltpu, tpu_sc as plsc

N_EDGES = 160000
DIM = 256
NSEG = 10000
HALF = DIM // 2

BLK = 1280
CH = 80
N_TILES = 16

BLKS_A = 63
BLKS_B = (N_EDGES // BLK) - BLKS_A
E_A = BLKS_A * BLK
E_B = N_EDGES - E_A
CHUNKS_A = E_A // N_TILES // CH
CHUNKS_B = E_B // N_TILES // CH

ZROWS_BIG = 640
ZROWS_LAST = NSEG - 15 * ZROWS_BIG


def _tc_body(x_ref, a_ref, wv_ref, c_ref, bv_ref, d_ref, o_ref):
    x = x_ref[...]
    xb = x.astype(jnp.bfloat16)
    t = jnp.dot(xb, a_ref[...], preferred_element_type=jnp.float32)
    v = lax.dot_general(xb, wv_ref[...], (((1,), (1,)), ((), ())),
                        preferred_element_type=jnp.float32)
    s = jnp.sum(x * (t + c_ref[...]), axis=1, keepdims=True) + d_ref[...]
    w = 1.0 / (1.0 + jnp.exp(-s))
    msg = (v + bv_ref[...]) * w
    o_ref[0] = msg[:, :HALF]
    o_ref[1] = msg[:, HALF:]


def _edge_messages(x, A, Wv, c, bv, d0, n_blocks, blk_off):
    return pl.pallas_call(
        _tc_body,
        grid=(n_blocks,),
        in_specs=[
            pl.BlockSpec((BLK, DIM), lambda i: (i + blk_off, 0)),
            pl.BlockSpec((DIM, DIM), lambda i: (0, 0)),
            pl.BlockSpec((DIM, DIM), lambda i: (0, 0)),
            pl.BlockSpec((1, DIM), lambda i: (0, 0)),
            pl.BlockSpec((1, DIM), lambda i: (0, 0)),
            pl.BlockSpec((1, 1), lambda i: (0, 0)),
        ],
        out_specs=pl.BlockSpec((2, BLK, HALF), lambda i: (0, i, 0)),
        out_shape=jax.ShapeDtypeStruct((2, n_blocks * BLK, HALF), jnp.float32),
    )(x, A, Wv, c, bv, d0)


def _sc_stage(msg_hbm, idx_hbm, src, out_hbm, idx_all, msg_v, acc,
              sem0, sem1, *, n_chunks, init_from_src):
    cid = lax.axis_index("c")
    sid = lax.axis_index("s")
    col0 = cid * HALF
    base_row = sid * ZROWS_BIG
    ept = n_chunks * CH
    base_e = cid * (N_TILES * ept) + sid * ept

    if init_from_src:
        @pl.when(sid < 15)
        def _():
            pltpu.sync_copy(
                src.at[pl.ds(base_row, ZROWS_BIG), pl.ds(col0, HALF)],
                acc.at[pl.ds(base_row, ZROWS_BIG), :])

        @pl.when(sid == 15)
        def _():
            pltpu.sync_copy(
                src.at[pl.ds(base_row, ZROWS_LAST), pl.ds(col0, HALF)],
                acc.at[pl.ds(base_row, ZROWS_LAST), :])
    else:
        def zero_vmem(i, _):
            r = i // (HALF // 16)
            c0 = (i % (HALF // 16)) * 16
            msg_v[0, r, pl.ds(c0, 16)] = jnp.zeros((16,), jnp.float32)
            return 0
        lax.fori_loop(0, CH * (HALF // 16), zero_vmem, 0)

        my_rows = jnp.where(sid < 15, ZROWS_BIG, ZROWS_LAST)

        def zero_acc(i, _):
            pltpu.sync_copy(msg_v.at[0], acc.at[pl.ds(base_row + i * CH, CH), :])
            return 0
        lax.fori_loop(0, my_rows // CH, zero_acc, 0)

    pltpu.sync_copy(idx_hbm.at[sid], idx_all)

    plsc.subcore_barrier()

    def start(j, buf, sem):
        return pltpu.async_copy(
            msg_hbm.at[pl.ds(base_e + j * CH, CH), :], msg_v.at[buf], sem)

    def wait(j, buf, sem):
        pltpu.make_async_copy(
            msg_hbm.at[pl.ds(base_e + j * CH, CH), :], msg_v.at[buf], sem
        ).wait()

    def scatter(j, buf):
        pltpu.sync_copy(msg_v.at[buf], acc.at[idx_all.at[j]], add=True)

    start(0, 0, sem0)

    def pair(p, _):
        j0 = 2 * p
        start(j0 + 1, 1, sem1)
        wait(j0, 0, sem0)
        scatter(j0, 0)
        start(j0 + 2, 0, sem0)
        wait(j0 + 1, 1, sem1)
        scatter(j0 + 1, 1)
        return 0
    lax.fori_loop(0, (n_chunks - 1) // 2, pair, 0)

    if n_chunks % 2 == 1:
        wait(n_chunks - 1, 0, sem0)
        scatter(n_chunks - 1, 0)
    else:
        wait(n_chunks - 2, 0, sem0)
        scatter(n_chunks - 2, 0)
        pltpu.sync_copy(
            msg_hbm.at[pl.ds(base_e + (n_chunks - 1) * CH, CH), :],
            msg_v.at[1])
        scatter(n_chunks - 1, 1)

    plsc.subcore_barrier()

    @pl.when(sid < 15)
    def _():
        pltpu.sync_copy(
            acc.at[pl.ds(base_row, ZROWS_BIG), :],
            out_hbm.at[pl.ds(base_row, ZROWS_BIG), pl.ds(col0, HALF)],
        )

    @pl.when(sid == 15)
    def _():
        pltpu.sync_copy(
            acc.at[pl.ds(base_row, ZROWS_LAST), :],
            out_hbm.at[pl.ds(base_row, ZROWS_LAST), pl.ds(col0, HALF)],
        )


def _segment_sum_stage(msg2, idx3d, partial, n_chunks):
    mesh = plsc.VectorSubcoreMesh(core_axis_name="c", subcore_axis_name="s")
    init_from_src = partial is not None
    scratch = [
        pltpu.VMEM((n_chunks, CH), jnp.int32),
        pltpu.VMEM((2, CH, HALF), jnp.float32),
        pltpu.VMEM_SHARED((NSEG, HALF), jnp.float32),
        pltpu.SemaphoreType.DMA,
        pltpu.SemaphoreType.DMA,
    ]
    body = functools.partial(_sc_stage, n_chunks=n_chunks,
                             init_from_src=init_from_src)

    if init_from_src:
        @functools.partial(
            pl.kernel, mesh=mesh,
            out_type=jax.ShapeDtypeStruct((NSEG, DIM), jnp.float32),
            scratch_types=scratch)
        def run(msg_hbm, idx_hbm, src_hbm, out_hbm, *rest):
            body(msg_hbm, idx_hbm, src_hbm, out_hbm, *rest)
        return run(msg2, idx3d, partial)
    else:
        @functools.partial(
            pl.kernel, mesh=mesh,
            out_type=jax.ShapeDtypeStruct((NSEG, DIM), jnp.float32),
            scratch_types=scratch)
        def run(msg_hbm, idx_hbm, out_hbm, *rest):
            body(msg_hbm, idx_hbm, None, out_hbm, *rest)
        return run(msg2, idx3d)


def kernel(x, index, dim_size, Wq, bq, Wk, bk, Wv, bv):
    A = lax.dot_general(Wq, Wk, (((0,), (0,)), ((), ())))
    c = (Wq.T @ bk + Wk.T @ bq).reshape(1, DIM)
    d0 = jnp.dot(bq, bk).reshape(1, 1)
    bv2 = bv.reshape(1, DIM)
    Ab = A.astype(jnp.bfloat16)
    Wvb = Wv.astype(jnp.bfloat16)

    idx = jnp.minimum(index, dim_size - 1).astype(jnp.int32)
    idxA = idx[:E_A].reshape(N_TILES, CHUNKS_A, CH)
    idxB = idx[E_A:].reshape(N_TILES, CHUNKS_B, CH)

    msgA = _edge_messages(x, Ab, Wvb, c, bv2, d0, BLKS_A, 0)
    partial = _segment_sum_stage(msgA.reshape(2 * E_A, HALF), idxA, None,
                                 CHUNKS_A)
    msgB = _edge_messages(x, Ab, Wvb, c, bv2, d0, BLKS_B, BLKS_A)
    out = _segment_sum_stage(msgB.reshape(2 * E_B, HALF), idxB, partial,
                             CHUNKS_B)
    return out


# speedup vs baseline: 1.4661x; 1.3101x over previous
import functools

import jax
import jax.numpy as jnp
from jax import lax
from jax.experimental import pallas as pl
from jax.experimental.pallas import tpu as pltpu, tpu_sc as plsc

N_EDGES = 160000
DIM = 256
NSEG = 10000
HALF = DIM // 2

BLK = 1280
CH = 80
N_TILES = 16

BLKS_A = 63
BLKS_B = (N_EDGES // BLK) - BLKS_A
E_A = BLKS_A * BLK
E_B = N_EDGES - E_A
CHUNKS_A = E_A // N_TILES // CH
CHUNKS_B = E_B // N_TILES // CH

ZROWS_BIG = 640
ZROWS_LAST = NSEG - 15 * ZROWS_BIG


def _tc_body(x_ref, a_ref, wv_ref, c_ref, bv_ref, d_ref, o_ref):
    x = x_ref[...]
    xb = x.astype(jnp.bfloat16)
    t = jnp.dot(xb, a_ref[...], preferred_element_type=jnp.float32)
    v = lax.dot_general(xb, wv_ref[...], (((1,), (1,)), ((), ())),
                        preferred_element_type=jnp.float32)
    y = (x * (t + c_ref[...])).astype(jnp.bfloat16)
    ones = jnp.ones((DIM, 8), jnp.bfloat16)
    s = jnp.dot(y, ones, preferred_element_type=jnp.float32)[:, :1]
    w = 1.0 / (1.0 + jnp.exp(-(s + d_ref[...])))
    msg = (v + bv_ref[...]) * w
    o_ref[0] = msg[:, :HALF]
    o_ref[1] = msg[:, HALF:]


def _edge_messages(x, A, Wv, c, bv, d0, n_blocks, blk_off):
    return pl.pallas_call(
        _tc_body,
        grid=(n_blocks,),
        in_specs=[
            pl.BlockSpec((BLK, DIM), lambda i: (i + blk_off, 0)),
            pl.BlockSpec((DIM, DIM), lambda i: (0, 0)),
            pl.BlockSpec((DIM, DIM), lambda i: (0, 0)),
            pl.BlockSpec((1, DIM), lambda i: (0, 0)),
            pl.BlockSpec((1, DIM), lambda i: (0, 0)),
            pl.BlockSpec((1, 1), lambda i: (0, 0)),
        ],
        out_specs=pl.BlockSpec((2, BLK, HALF), lambda i: (0, i, 0)),
        out_shape=jax.ShapeDtypeStruct((2, n_blocks * BLK, HALF), jnp.float32),
    )(x, A, Wv, c, bv, d0)


def _sc_stage(msg_hbm, idx_hbm, src, out_hbm, idx_all, msg_v, acc,
              sem0, sem1, *, n_chunks, init_from_src):
    cid = lax.axis_index("c")
    sid = lax.axis_index("s")
    col0 = cid * HALF
    base_row = sid * ZROWS_BIG
    ept = n_chunks * CH
    base_e = cid * (N_TILES * ept) + sid * ept

    if init_from_src:
        @pl.when(sid < 15)
        def _():
            pltpu.sync_copy(
                src.at[pl.ds(base_row, ZROWS_BIG), pl.ds(col0, HALF)],
                acc.at[pl.ds(base_row, ZROWS_BIG), :])

        @pl.when(sid == 15)
        def _():
            pltpu.sync_copy(
                src.at[pl.ds(base_row, ZROWS_LAST), pl.ds(col0, HALF)],
                acc.at[pl.ds(base_row, ZROWS_LAST), :])
    else:
        def zero_vmem(i, _):
            r = i // (HALF // 16)
            c0 = (i % (HALF // 16)) * 16
            msg_v[0, r, pl.ds(c0, 16)] = jnp.zeros((16,), jnp.float32)
            return 0
        lax.fori_loop(0, CH * (HALF // 16), zero_vmem, 0)

        my_rows = jnp.where(sid < 15, ZROWS_BIG, ZROWS_LAST)

        def zero_acc(i, _):
            pltpu.sync_copy(msg_v.at[0], acc.at[pl.ds(base_row + i * CH, CH), :])
            return 0
        lax.fori_loop(0, my_rows // CH, zero_acc, 0)

    pltpu.sync_copy(idx_hbm.at[sid], idx_all)

    plsc.subcore_barrier()

    def start(j, buf, sem):
        return pltpu.async_copy(
            msg_hbm.at[pl.ds(base_e + j * CH, CH), :], msg_v.at[buf], sem)

    def wait(j, buf, sem):
        pltpu.make_async_copy(
            msg_hbm.at[pl.ds(base_e + j * CH, CH), :], msg_v.at[buf], sem
        ).wait()

    def scatter(j, buf):
        pltpu.sync_copy(msg_v.at[buf], acc.at[idx_all.at[j]], add=True)

    start(0, 0, sem0)

    def pair(p, _):
        j0 = 2 * p
        start(j0 + 1, 1, sem1)
        wait(j0, 0, sem0)
        scatter(j0, 0)
        start(j0 + 2, 0, sem0)
        wait(j0 + 1, 1, sem1)
        scatter(j0 + 1, 1)
        return 0
    lax.fori_loop(0, (n_chunks - 1) // 2, pair, 0)

    if n_chunks % 2 == 1:
        wait(n_chunks - 1, 0, sem0)
        scatter(n_chunks - 1, 0)
    else:
        wait(n_chunks - 2, 0, sem0)
        scatter(n_chunks - 2, 0)
        pltpu.sync_copy(
            msg_hbm.at[pl.ds(base_e + (n_chunks - 1) * CH, CH), :],
            msg_v.at[1])
        scatter(n_chunks - 1, 1)

    plsc.subcore_barrier()

    @pl.when(sid < 15)
    def _():
        pltpu.sync_copy(
            acc.at[pl.ds(base_row, ZROWS_BIG), :],
            out_hbm.at[pl.ds(base_row, ZROWS_BIG), pl.ds(col0, HALF)],
        )

    @pl.when(sid == 15)
    def _():
        pltpu.sync_copy(
            acc.at[pl.ds(base_row, ZROWS_LAST), :],
            out_hbm.at[pl.ds(base_row, ZROWS_LAST), pl.ds(col0, HALF)],
        )


def _segment_sum_stage(msg2, idx3d, partial, n_chunks):
    mesh = plsc.VectorSubcoreMesh(core_axis_name="c", subcore_axis_name="s")
    init_from_src = partial is not None
    scratch = [
        pltpu.VMEM((n_chunks, CH), jnp.int32),
        pltpu.VMEM((2, CH, HALF), jnp.float32),
        pltpu.VMEM_SHARED((NSEG, HALF), jnp.float32),
        pltpu.SemaphoreType.DMA,
        pltpu.SemaphoreType.DMA,
    ]
    body = functools.partial(_sc_stage, n_chunks=n_chunks,
                             init_from_src=init_from_src)

    if init_from_src:
        @functools.partial(
            pl.kernel, mesh=mesh,
            out_type=jax.ShapeDtypeStruct((NSEG, DIM), jnp.float32),
            scratch_types=scratch)
        def run(msg_hbm, idx_hbm, src_hbm, out_hbm, *rest):
            body(msg_hbm, idx_hbm, src_hbm, out_hbm, *rest)
        return run(msg2, idx3d, partial)
    else:
        @functools.partial(
            pl.kernel, mesh=mesh,
            out_type=jax.ShapeDtypeStruct((NSEG, DIM), jnp.float32),
            scratch_types=scratch)
        def run(msg_hbm, idx_hbm, out_hbm, *rest):
            body(msg_hbm, idx_hbm, None, out_hbm, *rest)
        return run(msg2, idx3d)


def kernel(x, index, dim_size, Wq, bq, Wk, bk, Wv, bv):
    A = lax.dot_general(Wq, Wk, (((0,), (0,)), ((), ())))
    c = (Wq.T @ bk + Wk.T @ bq).reshape(1, DIM)
    d0 = jnp.dot(bq, bk).reshape(1, 1)
    bv2 = bv.reshape(1, DIM)
    Ab = A.astype(jnp.bfloat16)
    Wvb = Wv.astype(jnp.bfloat16)

    idx = jnp.minimum(index, dim_size - 1).astype(jnp.int32)
    idxA = idx[:E_A].reshape(N_TILES, CHUNKS_A, CH)
    idxB = idx[E_A:].reshape(N_TILES, CHUNKS_B, CH)

    msgA = _edge_messages(x, Ab, Wvb, c, bv2, d0, BLKS_A, 0)
    partial = _segment_sum_stage(msgA.reshape(2 * E_A, HALF), idxA, None,
                                 CHUNKS_A)
    msgB = _edge_messages(x, Ab, Wvb, c, bv2, d0, BLKS_B, BLKS_A)
    out = _segment_sum_stage(msgB.reshape(2 * E_B, HALF), idxB, partial,
                             CHUNKS_B)
    return out
